# baseline (speedup 1.0000x reference)
import functools
import math

import jax
import jax.numpy as jnp
from jax import lax
from jax.experimental import pallas as pl
from jax.experimental.pallas import tpu as pltpu
from jax.experimental.pallas import tpu_sc as plsc

_D = 512
_SCALE = math.sqrt(_D)
_NC, _NS = 2, 16
_NW = _NC * _NS
_CHUNK = 32
_NBUF = 3
_NSB = 3
_LANES = 16


def _make_scaled_gather(n, d):
    per_w = n // _NW
    n_chunks = per_w // _CHUNK
    mesh = plsc.VectorSubcoreMesh(
        core_axis_name="c", subcore_axis_name="s",
        num_cores=_NC, num_subcores=_NS)

    @functools.partial(
        pl.kernel,
        out_type=jax.ShapeDtypeStruct((n, d), jnp.float32),
        mesh=mesh,
        scratch_types=[
            pltpu.VMEM((per_w,), jnp.int32),
            *[pltpu.VMEM((_CHUNK, d), jnp.float32) for _ in range(_NBUF)],
            pltpu.VMEM_SHARED((_NS * _NSB, _CHUNK, d), jnp.float32),
            *[pltpu.SemaphoreType.DMA for _ in range(_NBUF + 2 * _NSB)],
        ],
    )
    def emb(x_hbm, lut_hbm, out_hbm, idx_v, *rest):
        rows = rest[:_NBUF]
        stage = rest[_NBUF]
        in_sems = rest[_NBUF + 1:2 * _NBUF + 1]
        xb_sems = rest[2 * _NBUF + 1:2 * _NBUF + 1 + _NSB]
        out_sems = rest[2 * _NBUF + 1 + _NSB:]
        cid = lax.axis_index("c")
        sid = lax.axis_index("s")
        wid = sid * _NC + cid
        base = wid * per_w
        pltpu.sync_copy(x_hbm.at[pl.ds(base, per_w)], idx_v)

        def start_gather(c):
            b = c % _NBUF
            return pltpu.async_copy(
                lut_hbm.at[idx_v.at[pl.ds(c * _CHUNK, _CHUNK)]],
                rows[b], in_sems[b])

        xbs = {}
        outs = {}

        def wait_xb(c):
            if c in xbs:
                xbs.pop(c).wait()

        gathers = {0: start_gather(0)}
        for c in range(n_chunks):
            b = c % _NBUF
            sb = c % _NSB
            nxt = c + 1
            if nxt < n_chunks:
                wait_xb(nxt - _NBUF)
                gathers[nxt] = start_gather(nxt)
            gathers.pop(c).wait()

            def row_body(r, acc, _b=b):
                for j in range(d // _LANES):
                    sl = (r, pl.ds(j * _LANES, _LANES))
                    rows[_b][sl] = rows[_b][sl] * _SCALE
                return acc
            lax.fori_loop(0, _CHUNK, row_body, 0)

            if c - _NSB in outs:
                outs.pop(c - _NSB).wait()
            xbs[c] = pltpu.async_copy(
                rows[b], stage.at[sid * _NSB + sb], xb_sems[sb])
            prev = c - 1
            if prev >= 0:
                wait_xb(prev)
                outs[prev] = pltpu.async_copy(
                    stage.at[sid * _NSB + prev % _NSB],
                    out_hbm.at[pl.ds(base + prev * _CHUNK, _CHUNK)],
                    out_sems[prev % _NSB])
        last = n_chunks - 1
        wait_xb(last)
        outs[last] = pltpu.async_copy(
            stage.at[sid * _NSB + last % _NSB],
            out_hbm.at[pl.ds(base + last * _CHUNK, _CHUNK)],
            out_sems[last % _NSB])
        for c in sorted(outs):
            outs.pop(c).wait()

    return emb


def kernel(x, lut):
    b, s = x.shape
    x_flat = x.reshape(-1).astype(jnp.int32)
    out = _make_scaled_gather(x_flat.shape[0], lut.shape[1])(x_flat, lut)
    return out.reshape(b, s, lut.shape[1])


# speedup vs baseline: 1.0402x; 1.0402x over previous
import functools
import math

import jax
import jax.numpy as jnp
from jax import lax
from jax.experimental import pallas as pl
from jax.experimental.pallas import tpu as pltpu
from jax.experimental.pallas import tpu_sc as plsc

_D = 512
_SCALE = math.sqrt(_D)
_NC, _NS = 2, 16
_NW = _NC * _NS
_CHUNK = 64
_NBUF = 3
_LANES = 16


def _make_scaled_gather(bsz, seq, d):
    n = bsz * seq
    per_w = n // _NW
    w_per_b = seq // per_w
    n_chunks = per_w // _CHUNK
    mesh = plsc.VectorSubcoreMesh(
        core_axis_name="c", subcore_axis_name="s",
        num_cores=_NC, num_subcores=_NS)

    @functools.partial(
        pl.kernel,
        out_type=jax.ShapeDtypeStruct((bsz, seq, d), jnp.float32),
        mesh=mesh,
        scratch_types=[
            pltpu.VMEM((per_w,), jnp.int32),
            *[pltpu.VMEM((_CHUNK, d), jnp.float32) for _ in range(_NBUF)],
            *[pltpu.SemaphoreType.DMA for _ in range(2 * _NBUF)],
        ],
    )
    def emb(x_hbm, lut_hbm, out_hbm, idx_v, *rest):
        rows = rest[:_NBUF]
        in_sems = rest[_NBUF:2 * _NBUF]
        out_sems = rest[2 * _NBUF:]
        wid = lax.axis_index("s") * _NC + lax.axis_index("c")
        b_i = wid // w_per_b
        col = (wid % w_per_b) * per_w
        pltpu.sync_copy(x_hbm.at[b_i, pl.ds(col, per_w)], idx_v)

        def start_gather(c):
            b = c % _NBUF
            return pltpu.async_copy(
                lut_hbm.at[idx_v.at[pl.ds(c * _CHUNK, _CHUNK)]],
                rows[b], in_sems[b])

        gathers = {0: start_gather(0)}
        stores = {}
        for c in range(n_chunks):
            b = c % _NBUF
            nxt = c + 1
            if nxt < n_chunks:
                if nxt - _NBUF in stores:
                    stores.pop(nxt - _NBUF).wait()
                gathers[nxt] = start_gather(nxt)
            gathers.pop(c).wait()

            def row_body(r, acc, _b=b):
                for j in range(d // _LANES):
                    sl = (r, pl.ds(j * _LANES, _LANES))
                    rows[_b][sl] = rows[_b][sl] * _SCALE
                return acc
            lax.fori_loop(0, _CHUNK, row_body, 0)

            stores[c] = pltpu.async_copy(
                rows[b],
                out_hbm.at[b_i, pl.ds(col + c * _CHUNK, _CHUNK)],
                out_sems[b])
        for c in sorted(stores):
            stores.pop(c).wait()

    return emb


def kernel(x, lut):
    bsz, seq = x.shape
    return _make_scaled_gather(bsz, seq, lut.shape[1])(
        x.astype(jnp.int32), lut)
